# baseline (speedup 1.0000x reference)
import functools

import jax
import jax.numpy as jnp
from jax import lax
from jax.experimental import pallas as pl
from jax.experimental.pallas import tpu as pltpu
from jax.experimental.pallas import tpu_sc as plsc

H = W = 2048
NCIN = 10
NCOUT = 8

_info = plsc.get_sparse_core_info()
NC, NS, L = _info.num_cores, _info.num_subcores, _info.num_lanes
NW = NC * NS
ROWS_PW = H // NW
GROUPS = W // 16


def _mag(dx, dy):
    ss = dx * dx + dy * dy
    i = lax.bitcast_convert_type(ss, jnp.int32)
    r = lax.bitcast_convert_type(jnp.int32(0x5F3759DF) - (i >> 1), jnp.float32)
    hs = 0.5 * ss
    for _ in range(2):
        r = r * (1.5 - hs * (r * r))
    return ss * r


def _compute_chunk(cos_v, grad_v, out_v):
    def body(g, carry):
        s = pl.ds(g * 16, 16)
        c = [cos_v[j, s] for j in range(NCOUT)]
        m01, m23 = jnp.maximum(c[0], c[1]), jnp.maximum(c[2], c[3])
        m45, m67 = jnp.maximum(c[4], c[5]), jnp.maximum(c[6], c[7])
        best = jnp.maximum(jnp.maximum(m01, m23), jnp.maximum(m45, m67))
        mag = _mag(grad_v[0, s], grad_v[1, s])
        for b in range(NCOUT):
            out_v[b, s] = jnp.where(c[b] == best, mag, 0.0)
        return carry

    lax.fori_loop(0, GROUPS, body, 0, unroll=8)


NIN = 3
NOUT = 2
STEP = 6


@functools.partial(
    pl.kernel,
    out_type=jax.ShapeDtypeStruct((1, NCOUT, H, W), jnp.float32),
    mesh=plsc.VectorSubcoreMesh(core_axis_name="c", subcore_axis_name="s"),
    scratch_types=[
        pltpu.VMEM((NCOUT, W), jnp.float32),
        pltpu.VMEM((NCOUT, W), jnp.float32),
        pltpu.VMEM((NCOUT, W), jnp.float32),
        pltpu.VMEM((2, W), jnp.float32),
        pltpu.VMEM((2, W), jnp.float32),
        pltpu.VMEM((2, W), jnp.float32),
        pltpu.VMEM((NCOUT, W), jnp.float32),
        pltpu.VMEM((NCOUT, W), jnp.float32),
        pltpu.SemaphoreType.DMA,
        pltpu.SemaphoreType.DMA,
        pltpu.SemaphoreType.DMA,
        pltpu.SemaphoreType.DMA,
        pltpu.SemaphoreType.DMA,
        pltpu.SemaphoreType.DMA,
        pltpu.SemaphoreType.DMA,
        pltpu.SemaphoreType.DMA,
    ],
)
def _hist_sc(x_hbm, out_hbm, cos_v0, cos_v1, cos_v2, grad_v0, grad_v1,
             grad_v2, out_v0, out_v1, csem0, csem1, csem2, gsem0, gsem1,
             gsem2, osem0, osem1):
    wid = lax.axis_index("s") * NC + lax.axis_index("c")
    row0 = wid * ROWS_PW

    cos_bufs = (cos_v0, cos_v1, cos_v2)
    grad_bufs = (grad_v0, grad_v1, grad_v2)
    out_bufs = (out_v0, out_v1)
    csems = (csem0, csem1, csem2)
    gsems = (gsem0, gsem1, gsem2)
    osems = (osem0, osem1)

    def start_in(ci, b):
        r = row0 + ci
        pltpu.async_copy(x_hbm.at[0, 0:NCOUT, r, :], cos_bufs[b], csems[b])
        pltpu.async_copy(x_hbm.at[0, NCOUT:NCIN, r, :], grad_bufs[b], gsems[b])

    def wait_in(b):
        pltpu.make_async_copy(
            x_hbm.at[0, 0:NCOUT, row0, :], cos_bufs[b], csems[b]).wait()
        pltpu.make_async_copy(
            x_hbm.at[0, NCOUT:NCIN, row0, :], grad_bufs[b], gsems[b]).wait()

    def start_out(ci, b):
        pltpu.async_copy(out_bufs[b], out_hbm.at[0, :, row0 + ci, :], osems[b])

    def wait_out(b):
        pltpu.make_async_copy(out_bufs[b], out_hbm.at[0, :, row0, :], osems[b]).wait()

    def compute(ib, ob):
        _compute_chunk(cos_bufs[ib], grad_bufs[ib], out_bufs[ob])

    for b in range(NIN):
        start_in(b, b)
    for ci in range(NIN):
        wait_in(ci % NIN)
        if ci >= NOUT:
            wait_out(ci % NOUT)
        compute(ci % NIN, ci % NOUT)
        start_out(ci, ci % NOUT)
        start_in(ci + NIN, ci % NIN)

    n_steady = (ROWS_PW - 2 * NIN + 1) // STEP
    tail0 = NIN + n_steady * STEP

    def sextet(p, carry):
        for j in range(STEP):
            ci = NIN + STEP * p + j
            wait_in(j % NIN)
            wait_out((NIN + j) % NOUT)
            compute(j % NIN, (NIN + j) % NOUT)
            start_out(ci, (NIN + j) % NOUT)
            start_in(ci + NIN, j % NIN)
        return carry

    lax.fori_loop(0, n_steady, sextet, 0)

    for ci in range(tail0, ROWS_PW):
        wait_in(ci % NIN)
        wait_out(ci % NOUT)
        compute(ci % NIN, ci % NOUT)
        start_out(ci, ci % NOUT)
        if ci + NIN < ROWS_PW:
            start_in(ci + NIN, ci % NIN)
    for b in range(NOUT):
        wait_out(b)


def kernel(x):
    return _hist_sc(x)


# speedup vs baseline: 1.4504x; 1.4504x over previous
import functools

import jax
import jax.numpy as jnp
from jax import lax
from jax.experimental import pallas as pl
from jax.experimental.pallas import tpu as pltpu
from jax.experimental.pallas import tpu_sc as plsc

H = W = 2048
NCIN = 10
NCOUT = 8

HSC = 1280
HTC = H - HSC
BH = 64

_info = plsc.get_sparse_core_info()
NC, NS, L = _info.num_cores, _info.num_subcores, _info.num_lanes
NW = NC * NS
ROWS_PW = HSC // NW
GROUPS = W // 16


def _mag(dx, dy):
    ss = dx * dx + dy * dy
    i = lax.bitcast_convert_type(ss, jnp.int32)
    r = lax.bitcast_convert_type(jnp.int32(0x5F3759DF) - (i >> 1), jnp.float32)
    hs = 0.5 * ss
    for _ in range(2):
        r = r * (1.5 - hs * (r * r))
    return ss * r


def _compute_chunk(in_v, out_v):
    def body(g, carry):
        s = pl.ds(g * 16, 16)
        c = [in_v[j, s] for j in range(NCOUT)]
        m01, m23 = jnp.maximum(c[0], c[1]), jnp.maximum(c[2], c[3])
        m45, m67 = jnp.maximum(c[4], c[5]), jnp.maximum(c[6], c[7])
        best = jnp.maximum(jnp.maximum(m01, m23), jnp.maximum(m45, m67))
        mag = _mag(in_v[8, s], in_v[9, s])
        for b in range(NCOUT):
            out_v[b, s] = jnp.where(c[b] == best, mag, 0.0)
        return carry

    lax.fori_loop(0, GROUPS, body, 0, unroll=8)


@functools.partial(
    pl.kernel,
    out_type=jax.ShapeDtypeStruct((1, NCOUT, H, W), jnp.float32),
    mesh=plsc.VectorSubcoreMesh(core_axis_name="c", subcore_axis_name="s"),
    scratch_types=[
        pltpu.VMEM((NCIN, W), jnp.float32),
        pltpu.VMEM((NCIN, W), jnp.float32),
        pltpu.VMEM((NCOUT, W), jnp.float32),
        pltpu.VMEM((NCOUT, W), jnp.float32),
        pltpu.SemaphoreType.DMA,
        pltpu.SemaphoreType.DMA,
        pltpu.SemaphoreType.DMA,
        pltpu.SemaphoreType.DMA,
    ],
)
def _hist_sc(x_hbm, out_hbm, in_v0, in_v1, out_v0, out_v1,
             isem0, isem1, osem0, osem1):
    wid = lax.axis_index("s") * NC + lax.axis_index("c")
    row0 = wid * ROWS_PW

    in_bufs = (in_v0, in_v1)
    out_bufs = (out_v0, out_v1)
    isems = (isem0, isem1)
    osems = (osem0, osem1)

    def start_in(ci, b):
        pltpu.async_copy(x_hbm.at[0, :, row0 + ci, :], in_bufs[b], isems[b])

    def wait_in(b):
        pltpu.make_async_copy(x_hbm.at[0, :, row0, :], in_bufs[b], isems[b]).wait()

    def start_out(ci, b):
        pltpu.async_copy(out_bufs[b], out_hbm.at[0, :, row0 + ci, :], osems[b])

    def wait_out(b):
        pltpu.make_async_copy(out_bufs[b], out_hbm.at[0, :, row0, :], osems[b]).wait()

    start_in(0, 0)
    start_in(1, 1)
    for b in range(2):
        wait_in(b)
        _compute_chunk(in_bufs[b], out_bufs[b])
        start_out(b, b)
        start_in(b + 2, b)

    def pair(p, carry):
        for b in range(2):
            ci = 2 * p + b
            wait_in(b)
            wait_out(b)
            _compute_chunk(in_bufs[b], out_bufs[b])
            start_out(ci, b)
            start_in(ci + 2, b)
        return carry

    lax.fori_loop(1, ROWS_PW // 2 - 1, pair, 0)

    for b in range(2):
        ci = ROWS_PW - 2 + b
        wait_in(b)
        wait_out(b)
        _compute_chunk(in_bufs[b], out_bufs[b])
        start_out(ci, b)
    for b in range(2):
        wait_out(b)


def _tc_body(x_ref, o_ref):
    xb = x_ref[0]
    cos = xb[:NCOUT]
    best = jnp.max(cos, axis=0, keepdims=True)
    mag = jnp.sqrt(xb[NCOUT] * xb[NCOUT] + xb[NCOUT + 1] * xb[NCOUT + 1])
    o_ref[0] = jnp.where(cos == best, mag[None], 0.0)


_hist_tc = pl.pallas_call(
    _tc_body,
    out_shape=jax.ShapeDtypeStruct((1, NCOUT, HTC, W), jnp.float32),
    grid=(HTC // BH,),
    in_specs=[pl.BlockSpec((1, NCIN, BH, W),
                           lambda i: (0, 0, HSC // BH + i, 0))],
    out_specs=pl.BlockSpec((1, NCOUT, BH, W), lambda i: (0, 0, i, 0)),
)


def kernel(x):
    full = _hist_sc(x)
    band = _hist_tc(x)
    return lax.dynamic_update_slice(full, band, (0, 0, HSC, 0))
